# Initial kernel scaffold; baseline (speedup 1.0000x reference)
import jax
import jax.numpy as jnp
from jax.experimental import pallas as pl


def kernel(x_claim, x_provider, edge_index_pc, edge_index_cp, edge_attr_pc, edge_attr_cp, params):
    raise NotImplementedError("write your pallas kernel here")


# speedup vs baseline: 8.2425x; 8.2425x over previous
import functools

import jax
import jax.numpy as jnp
from jax import lax
from jax.experimental import pallas as pl
from jax.experimental.pallas import tpu as pltpu
from jax.experimental.pallas import tpu_sc as plsc

F32 = jnp.float32
I32 = jnp.int32
C = 64
CH = 128


def _node_pre(x, scale, shift, Ws, avs, Wd, avd):
    N, din = x.shape
    B = 512
    nb = pl.cdiv(N, B)

    def body(x_ref, sc_ref, sh_ref, ws_ref, as_ref, wd_ref, ad_ref,
             hs_ref, asrc_ref, adst_ref):
        xn = x_ref[...] * sc_ref[...] + sh_ref[...]
        hs = jnp.dot(xn, ws_ref[...], preferred_element_type=F32)
        hs_ref[...] = hs
        asrc_ref[...] = jnp.sum(hs * as_ref[...], axis=1, keepdims=True)
        wv = jnp.sum(wd_ref[...] * ad_ref[...], axis=1, keepdims=True)
        adst_ref[...] = jnp.dot(xn, wv, preferred_element_type=F32)

    return pl.pallas_call(
        body,
        grid=(nb,),
        in_specs=[
            pl.BlockSpec((B, din), lambda i: (i, 0)),
            pl.BlockSpec((1, din), lambda i: (0, 0)),
            pl.BlockSpec((1, din), lambda i: (0, 0)),
            pl.BlockSpec((din, C), lambda i: (0, 0)),
            pl.BlockSpec((1, C), lambda i: (0, 0)),
            pl.BlockSpec((din, C), lambda i: (0, 0)),
            pl.BlockSpec((1, C), lambda i: (0, 0)),
        ],
        out_specs=[
            pl.BlockSpec((B, C), lambda i: (i, 0)),
            pl.BlockSpec((B, 1), lambda i: (i, 0)),
            pl.BlockSpec((B, 1), lambda i: (i, 0)),
        ],
        out_shape=[
            jax.ShapeDtypeStruct((N, C), F32),
            jax.ShapeDtypeStruct((N, 1), F32),
            jax.ShapeDtypeStruct((N, 1), F32),
        ],
    )(x, scale, shift, Ws, avs, Wd, avd)


def _edge_logits(ea2_a, ea2_b, We_a, ae_a, We_b, ae_b):
    R = ea2_a.shape[0]
    B = 512
    nb = pl.cdiv(R, B)

    def body(ea_ref, eb_ref, wa_ref, aa_ref, wb_ref, ab_ref, oa_ref, ob_ref):
        row = lax.broadcasted_iota(I32, (128, 8), 0)
        col = lax.broadcasted_iota(I32, (128, 8), 1)

        def one(e_ref, w_ref, a_ref, o_ref):
            wv = jnp.sum(w_ref[...] * a_ref[...], axis=1, keepdims=True)
            rep = jnp.concatenate([wv] * 8, axis=0)
            wt = jnp.where(row // 16 == col, rep, 0.0)
            o_ref[...] = jnp.dot(e_ref[...], wt, preferred_element_type=F32)

        one(ea_ref, wa_ref, aa_ref, oa_ref)
        one(eb_ref, wb_ref, ab_ref, ob_ref)

    return pl.pallas_call(
        body,
        grid=(nb,),
        in_specs=[
            pl.BlockSpec((B, 128), lambda i: (i, 0)),
            pl.BlockSpec((B, 128), lambda i: (i, 0)),
            pl.BlockSpec((16, C), lambda i: (0, 0)),
            pl.BlockSpec((1, C), lambda i: (0, 0)),
            pl.BlockSpec((16, C), lambda i: (0, 0)),
            pl.BlockSpec((1, C), lambda i: (0, 0)),
        ],
        out_specs=[
            pl.BlockSpec((B, 8), lambda i: (i, 0)),
            pl.BlockSpec((B, 8), lambda i: (i, 0)),
        ],
        out_shape=[
            jax.ShapeDtypeStruct((R, 8), F32),
            jax.ShapeDtypeStruct((R, 8), F32),
        ],
    )(ea2_a, ea2_b, We_a, ae_a, We_b, ae_b)


def _post_bn(num, den, b, g, beta, N):
    B = 512
    nb = pl.cdiv(N, B)

    def body(num_ref, den_ref, b_ref, g_ref, beta_ref, y_ref, ss_ref, acc_ref):
        i = pl.program_id(0)
        y = jnp.maximum(num_ref[...] / (den_ref[...] + 1e-16) + b_ref[...], 0.0)
        y_ref[...] = y
        rows = i * B + lax.broadcasted_iota(I32, (B, 1), 0)
        ym = jnp.where(rows < N, y, 0.0)

        @pl.when(i == 0)
        def _():
            acc_ref[...] = jnp.zeros_like(acc_ref)

        acc_ref[0:1, :] += jnp.sum(ym, axis=0, keepdims=True)
        acc_ref[1:2, :] += jnp.sum(ym * ym, axis=0, keepdims=True)

        @pl.when(i == nb - 1)
        def _():
            mean = acc_ref[0:1, :] * (1.0 / N)
            var = acc_ref[1:2, :] * (1.0 / N) - mean * mean
            sc = g_ref[...] * lax.rsqrt(var + 1e-5)
            sh = beta_ref[...] - mean * sc
            ss_ref[...] = jnp.concatenate([sc, sh], axis=0)

    return pl.pallas_call(
        body,
        grid=(nb,),
        in_specs=[
            pl.BlockSpec((B, C), lambda i: (i, 0)),
            pl.BlockSpec((B, 1), lambda i: (i, 0)),
            pl.BlockSpec((1, C), lambda i: (0, 0)),
            pl.BlockSpec((1, C), lambda i: (0, 0)),
            pl.BlockSpec((1, C), lambda i: (0, 0)),
        ],
        out_specs=[
            pl.BlockSpec((B, C), lambda i: (i, 0)),
            pl.BlockSpec((2, C), lambda i: (0, 0)),
        ],
        out_shape=[
            jax.ShapeDtypeStruct((N, C), F32),
            jax.ShapeDtypeStruct((2, C), F32),
        ],
        scratch_shapes=[pltpu.VMEM((2, C), F32)],
    )(num, den, b, g, beta)


def _head_a(y, ss, W1, b1, g, beta, N):
    B = 512
    nb = pl.cdiv(N, B)

    def body(y_ref, ss_ref, w_ref, b_ref, g_ref, beta_ref, z_ref, ss2_ref,
             acc_ref):
        i = pl.program_id(0)
        xn = y_ref[...] * ss_ref[0:1, :] + ss_ref[1:2, :]
        z = jnp.maximum(
            jnp.dot(xn, w_ref[...], preferred_element_type=F32) + b_ref[...],
            0.0)
        z_ref[...] = z
        rows = i * B + lax.broadcasted_iota(I32, (B, 1), 0)
        zm = jnp.where(rows < N, z, 0.0)

        @pl.when(i == 0)
        def _():
            acc_ref[...] = jnp.zeros_like(acc_ref)

        acc_ref[0:1, :] += jnp.sum(zm, axis=0, keepdims=True)
        acc_ref[1:2, :] += jnp.sum(zm * zm, axis=0, keepdims=True)

        @pl.when(i == nb - 1)
        def _():
            mean = acc_ref[0:1, :] * (1.0 / N)
            var = acc_ref[1:2, :] * (1.0 / N) - mean * mean
            sc = g_ref[...] * lax.rsqrt(var + 1e-5)
            sh = beta_ref[...] - mean * sc
            ss2_ref[...] = jnp.concatenate([sc, sh], axis=0)

    return pl.pallas_call(
        body,
        grid=(nb,),
        in_specs=[
            pl.BlockSpec((B, C), lambda i: (i, 0)),
            pl.BlockSpec((2, C), lambda i: (0, 0)),
            pl.BlockSpec((C, C), lambda i: (0, 0)),
            pl.BlockSpec((1, C), lambda i: (0, 0)),
            pl.BlockSpec((1, C), lambda i: (0, 0)),
            pl.BlockSpec((1, C), lambda i: (0, 0)),
        ],
        out_specs=[
            pl.BlockSpec((B, C), lambda i: (i, 0)),
            pl.BlockSpec((2, C), lambda i: (0, 0)),
        ],
        out_shape=[
            jax.ShapeDtypeStruct((N, C), F32),
            jax.ShapeDtypeStruct((2, C), F32),
        ],
        scratch_shapes=[pltpu.VMEM((2, C), F32)],
    )(y, ss, W1, b1, g, beta)


def _head_b(z, ss2, Wout, bout, N):
    B = 512
    nb = pl.cdiv(N, B)

    def body(z_ref, ss_ref, w_ref, b_ref, o_ref):
        xn = z_ref[...] * ss_ref[0:1, :] + ss_ref[1:2, :]
        t = jnp.dot(xn, w_ref[...], preferred_element_type=F32) + b_ref[...]
        o_ref[...] = jax.nn.sigmoid(t)

    return pl.pallas_call(
        body,
        grid=(nb,),
        in_specs=[
            pl.BlockSpec((B, C), lambda i: (i, 0)),
            pl.BlockSpec((2, C), lambda i: (0, 0)),
            pl.BlockSpec((C, 1), lambda i: (0, 0)),
            pl.BlockSpec((1, 1), lambda i: (0, 0)),
        ],
        out_specs=pl.BlockSpec((B, 1), lambda i: (i, 0)),
        out_shape=jax.ShapeDtypeStruct((N, 1), F32),
    )(z, ss2, Wout, bout)


def _sc_agg_body(zrows, hs, asrc, adst, aedge, src, dst, num_out, den_out,
                 asrc_t, adst_t, src_v, dl_v, ae_v, ex_v, rows_v, num_sh,
                 den_sh, sem, *, H, DELTA, R, EPT, NCHUNK):
    c = lax.axis_index("c")
    s = lax.axis_index("s")
    ZC = 32

    pltpu.sync_copy(asrc, asrc_t)
    pltpu.sync_copy(adst, adst_t)
    ebase = s * EPT

    for r in range(R):
        base_h = r * 2 * H + c * H

        pltpu.sync_copy(zrows, rows_v)
        for j in range(8):
            ex_v[pl.ds(16 * j, 16)] = jnp.zeros((16,), F32)
        for k in range(DELTA // ZC):
            r0 = s * DELTA + k * ZC
            pltpu.sync_copy(rows_v.at[pl.ds(0, ZC)], num_sh.at[pl.ds(r0, ZC)])
            pltpu.sync_copy(ex_v.at[pl.ds(0, ZC)], den_sh.at[pl.ds(r0, ZC)])
        plsc.subcore_barrier()

        def chunk(ci, _):
            off = ebase + ci * CH
            pltpu.sync_copy(src.at[pl.ds(off, CH)], src_v)
            pltpu.sync_copy(dst.at[pl.ds(off, CH)], dl_v)
            pltpu.sync_copy(aedge.at[pl.ds(off, CH)], ae_v)
            for j in range(8):
                sl = pl.ds(16 * j, 16)
                s16 = src_v[sl]
                d16 = dl_v[sl]
                av = plsc.load_gather(asrc_t, [s16])
                bv = plsc.load_gather(adst_t,
                                      [jnp.minimum(d16, adst_t.shape[0] - 1)])
                al = av + bv + ae_v[sl]
                al = jnp.where(al > 0, al, al * 0.2)
                ex_v[sl] = jnp.exp(al)
                dl = d16 - base_h
                ok = (dl >= 0) & (dl < H)
                dl_v[sl] = jnp.where(ok, dl, H)
            pltpu.sync_copy(ex_v, den_sh.at[dl_v], add=True)
            pltpu.async_copy(hs.at[src_v], rows_v, sem).wait()

            def srow(i, _):
                e = plsc.load_gather(ex_v, [jnp.full((16,), i, I32)])
                for k in range(4):
                    slk = pl.ds(16 * k, 16)
                    rows_v[i, slk] = rows_v[i, slk] * e
                return 0
            lax.fori_loop(0, CH, srow, 0)
            pltpu.sync_copy(rows_v, num_sh.at[dl_v], add=True)
            return 0

        lax.fori_loop(0, NCHUNK, chunk, 0)
        plsc.subcore_barrier()

        for k in range(DELTA // ZC):
            r0 = s * DELTA + k * ZC
            pltpu.sync_copy(num_sh.at[pl.ds(r0, ZC)],
                            num_out.at[pl.ds(base_h + r0, ZC)])
            pltpu.sync_copy(den_sh.at[pl.ds(r0, ZC)],
                            den_out.at[pl.ds(base_h + r0, ZC)])
        plsc.subcore_barrier()


def _sc_agg(zrows, hs, asrc, adst, aedge, src, dst, H, DELTA, R):
    Ns = asrc.shape[0]
    Nd = adst.shape[0]
    E_pad = src.shape[0]
    EPT = E_pad // 16
    NCHUNK = EPT // CH
    HB = H + 16
    mesh = plsc.VectorSubcoreMesh(core_axis_name="c", subcore_axis_name="s")
    kfn = pl.kernel(
        functools.partial(_sc_agg_body, H=H, DELTA=DELTA, R=R, EPT=EPT,
                          NCHUNK=NCHUNK),
        out_type=(
            jax.ShapeDtypeStruct((R * 2 * H, C), F32),
            jax.ShapeDtypeStruct((R * 2 * H,), F32),
        ),
        mesh=mesh,
        compiler_params=pltpu.CompilerParams(needs_layout_passes=False,
                                             use_tc_tiling_on_sc=False),
        scratch_types=[
            pltpu.VMEM((Ns,), F32),
            pltpu.VMEM((Nd,), F32),
            pltpu.VMEM((CH,), I32),
            pltpu.VMEM((CH,), I32),
            pltpu.VMEM((CH,), F32),
            pltpu.VMEM((CH,), F32),
            pltpu.VMEM((CH, C), F32),
            pltpu.VMEM_SHARED((HB, C), F32),
            pltpu.VMEM_SHARED((HB,), F32),
            pltpu.SemaphoreType.DMA,
        ],
    )
    return kfn(zrows, hs, asrc, adst, aedge, src, dst)


def _pad_to(x, n, fill):
    pad = n - x.shape[0]
    if pad == 0:
        return x
    return jnp.concatenate(
        [x, jnp.full((pad,) + x.shape[1:], fill, x.dtype)], axis=0)


def kernel(x_claim, x_provider, edge_index_pc, edge_index_cp, edge_attr_pc,
           edge_attr_cp, params):
    Nc, D = x_claim.shape
    Np = x_provider.shape[0]
    E = edge_index_pc.shape[1]
    E_pad = pl.cdiv(E, 16 * CH) * 16 * CH

    def _half(nd):
        h_max = 14848
        r = pl.cdiv(nd, 2 * h_max)
        h = pl.cdiv(nd, 2 * r * 512) * 512
        return h, h // 16, r

    H_c, DEL_c, R_c = _half(Nc)
    H_p, DEL_p, R_p = _half(Np)

    src_pc = _pad_to(edge_index_pc[0], E_pad, 0)
    dst_pc = _pad_to(edge_index_pc[1], E_pad, 1 << 28)
    src_cp = _pad_to(edge_index_cp[0], E_pad, 0)
    dst_cp = _pad_to(edge_index_cp[1], E_pad, 1 << 28)
    ea2_pc = edge_attr_pc.reshape(E // 8, 128)
    ea2_cp = edge_attr_cp.reshape(E // 8, 128)

    ones_d = jnp.ones((1, D), F32)
    zeros_d = jnp.zeros((1, D), F32)
    zrows = jnp.zeros((CH, C), F32)
    xc, xp = x_claim, x_provider
    ss_c = None
    ss_p = None

    for l in range(2):
        ppc = params[f'pc{l}']
        pcp = params[f'cp{l}']
        r2 = lambda a: a.reshape(1, -1)
        if l == 0:
            sc_c, sh_c = ones_d, zeros_d
            sc_p, sh_p = ones_d, zeros_d
        else:
            sc_c, sh_c = ss_c[0:1, :], ss_c[1:2, :]
            sc_p, sh_p = ss_p[0:1, :], ss_p[1:2, :]

        hs_cp, asrc_cp, adst_pc = _node_pre(
            xc, sc_c, sh_c, pcp['Ws'], r2(pcp['as']), ppc['Wd'], r2(ppc['ad']))
        hs_pc, asrc_pc, adst_cp = _node_pre(
            xp, sc_p, sh_p, ppc['Ws'], r2(ppc['as']), pcp['Wd'], r2(pcp['ad']))

        ae_pc8, ae_cp8 = _edge_logits(
            ea2_pc, ea2_cp, ppc['We'], r2(ppc['ae']), pcp['We'], r2(pcp['ae']))
        aed_pc = _pad_to(ae_pc8.reshape(E), E_pad, 0.0)
        aed_cp = _pad_to(ae_cp8.reshape(E), E_pad, 0.0)

        num_c, den_c = _sc_agg(zrows, hs_pc, asrc_pc.reshape(Np),
                               adst_pc.reshape(Nc), aed_pc, src_pc, dst_pc,
                               H_c, DEL_c, R_c)
        num_p, den_p = _sc_agg(zrows, hs_cp, asrc_cp.reshape(Nc),
                               adst_cp.reshape(Np), aed_cp, src_cp, dst_cp,
                               H_p, DEL_p, R_p)

        bnc = params[f'bn_claim{l}']
        bnp = params[f'bn_prov{l}']
        xc, ss_c = _post_bn(num_c, den_c.reshape(R_c * 2 * H_c, 1),
                            r2(ppc['b']), r2(bnc['g']), r2(bnc['b']), Nc)
        xp, ss_p = _post_bn(num_p, den_p.reshape(R_p * 2 * H_p, 1),
                            r2(pcp['b']), r2(bnp['g']), r2(bnp['b']), Np)

    z, ss2 = _head_a(xc, ss_c, params['lin1']['W'],
                     params['lin1']['b'].reshape(1, C),
                     params['bn_lin']['g'].reshape(1, C),
                     params['bn_lin']['b'].reshape(1, C), Nc)
    return _head_b(z, ss2, params['out']['W'],
                   params['out']['b'].reshape(1, 1), Nc)


# speedup vs baseline: 9.5315x; 1.1564x over previous
import functools

import jax
import jax.numpy as jnp
from jax import lax
from jax.experimental import pallas as pl
from jax.experimental.pallas import tpu as pltpu
from jax.experimental.pallas import tpu_sc as plsc

F32 = jnp.float32
I32 = jnp.int32
C = 64
CH = 128


def _node_pre(x, scale, shift, Ws, avs, Wd, avd):
    N, din = x.shape
    B = 512
    nb = pl.cdiv(N, B)

    def body(x_ref, sc_ref, sh_ref, ws_ref, as_ref, wd_ref, ad_ref,
             hs_ref, asrc_ref, adst_ref):
        xn = x_ref[...] * sc_ref[...] + sh_ref[...]
        hs = jnp.dot(xn, ws_ref[...], preferred_element_type=F32)
        hs_ref[...] = hs
        asrc_ref[...] = jnp.sum(hs * as_ref[...], axis=1, keepdims=True)
        wv = jnp.sum(wd_ref[...] * ad_ref[...], axis=1, keepdims=True)
        adst_ref[...] = jnp.dot(xn, wv, preferred_element_type=F32)

    return pl.pallas_call(
        body,
        grid=(nb,),
        in_specs=[
            pl.BlockSpec((B, din), lambda i: (i, 0)),
            pl.BlockSpec((1, din), lambda i: (0, 0)),
            pl.BlockSpec((1, din), lambda i: (0, 0)),
            pl.BlockSpec((din, C), lambda i: (0, 0)),
            pl.BlockSpec((1, C), lambda i: (0, 0)),
            pl.BlockSpec((din, C), lambda i: (0, 0)),
            pl.BlockSpec((1, C), lambda i: (0, 0)),
        ],
        out_specs=[
            pl.BlockSpec((B, C), lambda i: (i, 0)),
            pl.BlockSpec((B, 1), lambda i: (i, 0)),
            pl.BlockSpec((B, 1), lambda i: (i, 0)),
        ],
        out_shape=[
            jax.ShapeDtypeStruct((N, C), F32),
            jax.ShapeDtypeStruct((N, 1), F32),
            jax.ShapeDtypeStruct((N, 1), F32),
        ],
    )(x, scale, shift, Ws, avs, Wd, avd)


def _edge_logits(ea2_a, ea2_b, We_a, ae_a, We_b, ae_b):
    R = ea2_a.shape[0]
    B = 512
    nb = pl.cdiv(R, B)

    def body(ea_ref, eb_ref, wa_ref, aa_ref, wb_ref, ab_ref, oa_ref, ob_ref):
        row = lax.broadcasted_iota(I32, (128, 8), 0)
        col = lax.broadcasted_iota(I32, (128, 8), 1)

        def one(e_ref, w_ref, a_ref, o_ref):
            wv = jnp.sum(w_ref[...] * a_ref[...], axis=1, keepdims=True)
            rep = jnp.concatenate([wv] * 8, axis=0)
            wt = jnp.where(row // 16 == col, rep, 0.0)
            o_ref[...] = jnp.dot(e_ref[...], wt, preferred_element_type=F32)

        one(ea_ref, wa_ref, aa_ref, oa_ref)
        one(eb_ref, wb_ref, ab_ref, ob_ref)

    return pl.pallas_call(
        body,
        grid=(nb,),
        in_specs=[
            pl.BlockSpec((B, 128), lambda i: (i, 0)),
            pl.BlockSpec((B, 128), lambda i: (i, 0)),
            pl.BlockSpec((16, C), lambda i: (0, 0)),
            pl.BlockSpec((1, C), lambda i: (0, 0)),
            pl.BlockSpec((16, C), lambda i: (0, 0)),
            pl.BlockSpec((1, C), lambda i: (0, 0)),
        ],
        out_specs=[
            pl.BlockSpec((B, 8), lambda i: (i, 0)),
            pl.BlockSpec((B, 8), lambda i: (i, 0)),
        ],
        out_shape=[
            jax.ShapeDtypeStruct((R, 8), F32),
            jax.ShapeDtypeStruct((R, 8), F32),
        ],
    )(ea2_a, ea2_b, We_a, ae_a, We_b, ae_b)


def _post_bn(num, den, b, g, beta, N):
    B = 512
    nb = pl.cdiv(N, B)

    def body(num_ref, den_ref, b_ref, g_ref, beta_ref, y_ref, ss_ref, acc_ref):
        i = pl.program_id(0)
        y = jnp.maximum(num_ref[...] / (den_ref[...] + 1e-16) + b_ref[...], 0.0)
        y_ref[...] = y
        rows = i * B + lax.broadcasted_iota(I32, (B, 1), 0)
        ym = jnp.where(rows < N, y, 0.0)

        @pl.when(i == 0)
        def _():
            acc_ref[...] = jnp.zeros_like(acc_ref)

        acc_ref[0:1, :] += jnp.sum(ym, axis=0, keepdims=True)
        acc_ref[1:2, :] += jnp.sum(ym * ym, axis=0, keepdims=True)

        @pl.when(i == nb - 1)
        def _():
            mean = acc_ref[0:1, :] * (1.0 / N)
            var = acc_ref[1:2, :] * (1.0 / N) - mean * mean
            sc = g_ref[...] * lax.rsqrt(var + 1e-5)
            sh = beta_ref[...] - mean * sc
            ss_ref[...] = jnp.concatenate([sc, sh], axis=0)

    return pl.pallas_call(
        body,
        grid=(nb,),
        in_specs=[
            pl.BlockSpec((B, C), lambda i: (i, 0)),
            pl.BlockSpec((B, 1), lambda i: (i, 0)),
            pl.BlockSpec((1, C), lambda i: (0, 0)),
            pl.BlockSpec((1, C), lambda i: (0, 0)),
            pl.BlockSpec((1, C), lambda i: (0, 0)),
        ],
        out_specs=[
            pl.BlockSpec((B, C), lambda i: (i, 0)),
            pl.BlockSpec((2, C), lambda i: (0, 0)),
        ],
        out_shape=[
            jax.ShapeDtypeStruct((N, C), F32),
            jax.ShapeDtypeStruct((2, C), F32),
        ],
        scratch_shapes=[pltpu.VMEM((2, C), F32)],
    )(num, den, b, g, beta)


def _head_a(y, ss, W1, b1, g, beta, N):
    B = 512
    nb = pl.cdiv(N, B)

    def body(y_ref, ss_ref, w_ref, b_ref, g_ref, beta_ref, z_ref, ss2_ref,
             acc_ref):
        i = pl.program_id(0)
        xn = y_ref[...] * ss_ref[0:1, :] + ss_ref[1:2, :]
        z = jnp.maximum(
            jnp.dot(xn, w_ref[...], preferred_element_type=F32) + b_ref[...],
            0.0)
        z_ref[...] = z
        rows = i * B + lax.broadcasted_iota(I32, (B, 1), 0)
        zm = jnp.where(rows < N, z, 0.0)

        @pl.when(i == 0)
        def _():
            acc_ref[...] = jnp.zeros_like(acc_ref)

        acc_ref[0:1, :] += jnp.sum(zm, axis=0, keepdims=True)
        acc_ref[1:2, :] += jnp.sum(zm * zm, axis=0, keepdims=True)

        @pl.when(i == nb - 1)
        def _():
            mean = acc_ref[0:1, :] * (1.0 / N)
            var = acc_ref[1:2, :] * (1.0 / N) - mean * mean
            sc = g_ref[...] * lax.rsqrt(var + 1e-5)
            sh = beta_ref[...] - mean * sc
            ss2_ref[...] = jnp.concatenate([sc, sh], axis=0)

    return pl.pallas_call(
        body,
        grid=(nb,),
        in_specs=[
            pl.BlockSpec((B, C), lambda i: (i, 0)),
            pl.BlockSpec((2, C), lambda i: (0, 0)),
            pl.BlockSpec((C, C), lambda i: (0, 0)),
            pl.BlockSpec((1, C), lambda i: (0, 0)),
            pl.BlockSpec((1, C), lambda i: (0, 0)),
            pl.BlockSpec((1, C), lambda i: (0, 0)),
        ],
        out_specs=[
            pl.BlockSpec((B, C), lambda i: (i, 0)),
            pl.BlockSpec((2, C), lambda i: (0, 0)),
        ],
        out_shape=[
            jax.ShapeDtypeStruct((N, C), F32),
            jax.ShapeDtypeStruct((2, C), F32),
        ],
        scratch_shapes=[pltpu.VMEM((2, C), F32)],
    )(y, ss, W1, b1, g, beta)


def _head_b(z, ss2, Wout, bout, N):
    B = 512
    nb = pl.cdiv(N, B)

    def body(z_ref, ss_ref, w_ref, b_ref, o_ref):
        xn = z_ref[...] * ss_ref[0:1, :] + ss_ref[1:2, :]
        t = jnp.dot(xn, w_ref[...], preferred_element_type=F32) + b_ref[...]
        o_ref[...] = jax.nn.sigmoid(t)

    return pl.pallas_call(
        body,
        grid=(nb,),
        in_specs=[
            pl.BlockSpec((B, C), lambda i: (i, 0)),
            pl.BlockSpec((2, C), lambda i: (0, 0)),
            pl.BlockSpec((C, 1), lambda i: (0, 0)),
            pl.BlockSpec((1, 1), lambda i: (0, 0)),
        ],
        out_specs=pl.BlockSpec((B, 1), lambda i: (i, 0)),
        out_shape=jax.ShapeDtypeStruct((N, 1), F32),
    )(z, ss2, Wout, bout)


def _sc_agg_body(zrows, hs, asrc, adst, aedge, src, dst, num_out, den_out,
                 asrc_t, adst_t, src_v0, src_v1, dl_v0, dl_v1, ae_v0, ae_v1,
                 ex_v0, ex_v1, rows_v0, rows_v1, num_sh, den_sh, sem0, sem1,
                 *, H, DELTA, R, EPT, NCHUNK):
    src_v = (src_v0, src_v1)
    dl_v = (dl_v0, dl_v1)
    ae_v = (ae_v0, ae_v1)
    ex_v = (ex_v0, ex_v1)
    rows_v = (rows_v0, rows_v1)
    sem = (sem0, sem1)
    c = lax.axis_index("c")
    s = lax.axis_index("s")
    nd_max = adst.shape[0] - 1

    pltpu.sync_copy(asrc, asrc_t)
    pltpu.sync_copy(adst, adst_t)
    ebase = s * EPT

    for r in range(R):
        base_h = r * 2 * H + c * H

        pltpu.sync_copy(zrows, rows_v[0])
        for j in range(8):
            ex_v[0][pl.ds(16 * j, 16)] = jnp.zeros((16,), F32)
        for r0 in range(0, DELTA, CH):
            n = min(CH, DELTA - r0)
            d0 = s * DELTA + r0
            pltpu.sync_copy(rows_v[0].at[pl.ds(0, n)],
                            num_sh.at[pl.ds(d0, n)])
            pltpu.sync_copy(ex_v[0].at[pl.ds(0, n)],
                            den_sh.at[pl.ds(d0, n)])
        plsc.subcore_barrier()

        def prep(ci, b):
            off = ebase + ci * CH
            pltpu.sync_copy(src.at[pl.ds(off, CH)], src_v[b])
            pltpu.sync_copy(dst.at[pl.ds(off, CH)], dl_v[b])
            pltpu.sync_copy(aedge.at[pl.ds(off, CH)], ae_v[b])
            for j in range(8):
                sl = pl.ds(16 * j, 16)
                s16 = src_v[b][sl]
                d16 = dl_v[b][sl]
                av = plsc.load_gather(asrc_t, [s16])
                bv = plsc.load_gather(adst_t, [jnp.minimum(d16, nd_max)])
                al = av + bv + ae_v[b][sl]
                al = jnp.where(al > 0, al, al * 0.2)
                ex_v[b][sl] = jnp.exp(al)
                dl = d16 - base_h
                ok = (dl >= 0) & (dl < H)
                dl_v[b][sl] = jnp.where(ok, dl, H)
            pltpu.sync_copy(ex_v[b], den_sh.at[dl_v[b]], add=True)
            pltpu.async_copy(hs.at[src_v[b]], rows_v[b], sem[b])

        def proc(b):
            pltpu.make_async_copy(hs.at[src_v[b]], rows_v[b], sem[b]).wait()

            def srow(i, _):
                e = plsc.load_gather(ex_v[b], [jnp.full((16,), i, I32)])
                for k in range(4):
                    slk = pl.ds(16 * k, 16)
                    rows_v[b][i, slk] = rows_v[b][i, slk] * e
                return 0
            lax.fori_loop(0, CH, srow, 0, unroll=4)
            pltpu.sync_copy(rows_v[b], num_sh.at[dl_v[b]], add=True)

        prep(0, 0)

        def pair(g2, _):
            for b in range(2):
                ci = g2 * 2 + b

                @pl.when(ci + 1 < NCHUNK)
                def _():
                    prep(ci + 1, 1 - b)
                proc(b)
            return 0

        lax.fori_loop(0, NCHUNK // 2, pair, 0)
        plsc.subcore_barrier()

        for r0 in range(0, DELTA, CH):
            n = min(CH, DELTA - r0)
            d0 = s * DELTA + r0
            pltpu.sync_copy(num_sh.at[pl.ds(d0, n)],
                            num_out.at[pl.ds(base_h + d0, n)])
            pltpu.sync_copy(den_sh.at[pl.ds(d0, n)],
                            den_out.at[pl.ds(base_h + d0, n)])
        plsc.subcore_barrier()


def _sc_agg(zrows, hs, asrc, adst, aedge, src, dst, H, DELTA, R):
    Ns = asrc.shape[0]
    Nd = adst.shape[0]
    E_pad = src.shape[0]
    EPT = E_pad // 16
    NCHUNK = EPT // CH
    HB = H + 16
    mesh = plsc.VectorSubcoreMesh(core_axis_name="c", subcore_axis_name="s")
    kfn = pl.kernel(
        functools.partial(_sc_agg_body, H=H, DELTA=DELTA, R=R, EPT=EPT,
                          NCHUNK=NCHUNK),
        out_type=(
            jax.ShapeDtypeStruct((R * 2 * H, C), F32),
            jax.ShapeDtypeStruct((R * 2 * H,), F32),
        ),
        mesh=mesh,
        compiler_params=pltpu.CompilerParams(needs_layout_passes=False,
                                             use_tc_tiling_on_sc=False),
        scratch_types=(
            [pltpu.VMEM((Ns,), F32),
             pltpu.VMEM((Nd,), F32)]
            + [pltpu.VMEM((CH,), I32)] * 2
            + [pltpu.VMEM((CH,), I32)] * 2
            + [pltpu.VMEM((CH,), F32)] * 2
            + [pltpu.VMEM((CH,), F32)] * 2
            + [pltpu.VMEM((CH, C), F32)] * 2
            + [pltpu.VMEM_SHARED((HB, C), F32),
               pltpu.VMEM_SHARED((HB,), F32),
               pltpu.SemaphoreType.DMA,
               pltpu.SemaphoreType.DMA]
        ),
    )
    return kfn(zrows, hs, asrc, adst, aedge, src, dst)


def _pad_to(x, n, fill):
    pad = n - x.shape[0]
    if pad == 0:
        return x
    return jnp.concatenate(
        [x, jnp.full((pad,) + x.shape[1:], fill, x.dtype)], axis=0)


def kernel(x_claim, x_provider, edge_index_pc, edge_index_cp, edge_attr_pc,
           edge_attr_cp, params):
    Nc, D = x_claim.shape
    Np = x_provider.shape[0]
    E = edge_index_pc.shape[1]
    E_pad = pl.cdiv(E, 32 * CH) * 32 * CH

    def _half(nd):
        h_max = 14848
        r = pl.cdiv(nd, 2 * h_max)
        h = pl.cdiv(nd, 2 * r * 512) * 512
        return h, h // 16, r

    H_c, DEL_c, R_c = _half(Nc)
    H_p, DEL_p, R_p = _half(Np)

    src_pc = _pad_to(edge_index_pc[0], E_pad, 0)
    dst_pc = _pad_to(edge_index_pc[1], E_pad, 1 << 28)
    src_cp = _pad_to(edge_index_cp[0], E_pad, 0)
    dst_cp = _pad_to(edge_index_cp[1], E_pad, 1 << 28)
    ea2_pc = edge_attr_pc.reshape(E // 8, 128)
    ea2_cp = edge_attr_cp.reshape(E // 8, 128)

    ones_d = jnp.ones((1, D), F32)
    zeros_d = jnp.zeros((1, D), F32)
    zrows = jnp.zeros((CH, C), F32)
    xc, xp = x_claim, x_provider
    ss_c = None
    ss_p = None

    for l in range(2):
        ppc = params[f'pc{l}']
        pcp = params[f'cp{l}']
        r2 = lambda a: a.reshape(1, -1)
        if l == 0:
            sc_c, sh_c = ones_d, zeros_d
            sc_p, sh_p = ones_d, zeros_d
        else:
            sc_c, sh_c = ss_c[0:1, :], ss_c[1:2, :]
            sc_p, sh_p = ss_p[0:1, :], ss_p[1:2, :]

        hs_cp, asrc_cp, adst_pc = _node_pre(
            xc, sc_c, sh_c, pcp['Ws'], r2(pcp['as']), ppc['Wd'], r2(ppc['ad']))
        hs_pc, asrc_pc, adst_cp = _node_pre(
            xp, sc_p, sh_p, ppc['Ws'], r2(ppc['as']), pcp['Wd'], r2(pcp['ad']))

        ae_pc8, ae_cp8 = _edge_logits(
            ea2_pc, ea2_cp, ppc['We'], r2(ppc['ae']), pcp['We'], r2(pcp['ae']))
        aed_pc = _pad_to(ae_pc8.reshape(E), E_pad, 0.0)
        aed_cp = _pad_to(ae_cp8.reshape(E), E_pad, 0.0)

        num_c, den_c = _sc_agg(zrows, hs_pc, asrc_pc.reshape(Np),
                               adst_pc.reshape(Nc), aed_pc, src_pc, dst_pc,
                               H_c, DEL_c, R_c)
        num_p, den_p = _sc_agg(zrows, hs_cp, asrc_cp.reshape(Nc),
                               adst_cp.reshape(Np), aed_cp, src_cp, dst_cp,
                               H_p, DEL_p, R_p)

        bnc = params[f'bn_claim{l}']
        bnp = params[f'bn_prov{l}']
        xc, ss_c = _post_bn(num_c, den_c.reshape(R_c * 2 * H_c, 1),
                            r2(ppc['b']), r2(bnc['g']), r2(bnc['b']), Nc)
        xp, ss_p = _post_bn(num_p, den_p.reshape(R_p * 2 * H_p, 1),
                            r2(pcp['b']), r2(bnp['g']), r2(bnp['b']), Np)

    z, ss2 = _head_a(xc, ss_c, params['lin1']['W'],
                     params['lin1']['b'].reshape(1, C),
                     params['bn_lin']['g'].reshape(1, C),
                     params['bn_lin']['b'].reshape(1, C), Nc)
    return _head_b(z, ss2, params['out']['W'],
                   params['out']['b'].reshape(1, 1), Nc)
